# baseline (speedup 1.0000x reference)
import functools

import jax
import jax.numpy as jnp
from jax import lax
from jax.experimental import pallas as pl
from jax.experimental.pallas import tpu as pltpu
from jax.experimental.pallas import tpu_sc as plsc

B = 16384
S = 50
EMB = 128
HID = 256

NC, NS = 2, 16
NW = NC * NS
EPW = B // NW
G = 2
NGROUPS = EPW // G
GPW = NGROUPS
NVR = EMB // 16
RU = 10
NBUF = 4
DEPTH = 3


def _sc_pool_body(idx_hbm, table_hbm, out_hbm, idx_all, *scratch):
    rows = scratch[0:NBUF]
    outs = scratch[NBUF:2 * NBUF]
    sgs = scratch[2 * NBUF:3 * NBUF]
    sos = scratch[3 * NBUF:4 * NBUF]
    wid = lax.axis_index("s") * NC + lax.axis_index("c")
    ebase = wid * EPW
    pltpu.sync_copy(idx_hbm.at[pl.ds(wid * GPW, GPW)], idx_all)

    def prefetch(g, p):
        pltpu.async_copy(table_hbm.at[idx_all.at[g]], rows[p], sgs[p])

    def consume(i, g, p):
        pltpu.make_async_copy(
            table_hbm.at[idx_all.at[g]], rows[p], sgs[p]).wait()

        @pl.when(i >= 1)
        def _():
            pltpu.make_async_copy(
                outs[p], out_hbm.at[pl.ds(0, G)], sos[p]).wait()

        for e in range(G):
            def row_body(rr, accs, e=e):
                base = e * S + rr * RU
                for k in range(RU):
                    accs = tuple(
                        accs[v] + rows[p][base + k, pl.ds(16 * v, 16)]
                        for v in range(NVR))
                return accs
            accs = lax.fori_loop(
                0, S // RU, row_body,
                tuple(jnp.zeros((16,), jnp.float32) for _ in range(NVR)))
            for v in range(NVR):
                outs[p][e, pl.ds(16 * v, 16)] = accs[v] * (1.0 / S)
        pltpu.async_copy(outs[p], out_hbm.at[pl.ds(ebase + g * G, G)], sos[p])

    for d in range(DEPTH):
        prefetch(d, d)

    def block(i, carry):
        for p in range(NBUF):
            g = NBUF * i + p

            @pl.when(g + DEPTH < NGROUPS)
            def _(p=p, g=g):
                prefetch(g + DEPTH, (p + DEPTH) % NBUF)

            consume(i, g, p)
        return carry

    lax.fori_loop(0, NGROUPS // NBUF, block, 0)
    for p in range(NBUF):
        pltpu.make_async_copy(outs[p], out_hbm.at[pl.ds(0, G)], sos[p]).wait()


_sc_pool = pl.kernel(
    _sc_pool_body,
    out_type=jax.ShapeDtypeStruct((B, EMB), jnp.float32),
    mesh=plsc.VectorSubcoreMesh(core_axis_name="c", subcore_axis_name="s"),
    scratch_types=(
        [pltpu.VMEM((GPW, G * S), jnp.int32)]
        + [pltpu.VMEM((G * S, EMB), jnp.float32) for _ in range(NBUF)]
        + [pltpu.VMEM((G, EMB), jnp.float32) for _ in range(NBUF)]
        + [pltpu.SemaphoreType.DMA for _ in range(2 * NBUF)]
    ),
)


def _mlp_body(x_ref, w1_ref, b1_ref, w2_ref, b2_ref, o_ref):
    x = x_ref[...]
    h = jnp.dot(x, w1_ref[...], preferred_element_type=jnp.float32)
    h = jnp.maximum(h + b1_ref[...], 0.0)
    o = jnp.dot(h, w2_ref[...], preferred_element_type=jnp.float32)
    o_ref[...] = jax.nn.sigmoid(o + b2_ref[...])


def _mlp(x, w1, b1, w2, b2):
    BM = 2048
    grid = (B // BM,)
    return pl.pallas_call(
        _mlp_body,
        out_shape=jax.ShapeDtypeStruct((B, 128), jnp.float32),
        grid=grid,
        in_specs=[
            pl.BlockSpec((BM, EMB), lambda i: (i, 0)),
            pl.BlockSpec((EMB, HID), lambda i: (0, 0)),
            pl.BlockSpec((1, HID), lambda i: (0, 0)),
            pl.BlockSpec((HID, 128), lambda i: (0, 0)),
            pl.BlockSpec((1, 128), lambda i: (0, 0)),
        ],
        out_specs=pl.BlockSpec((BM, 128), lambda i: (i, 0)),
    )(x, w1, b1, w2, b2)


def kernel(inputs, table, W1, b1, W2, b2):
    idx = inputs.astype(jnp.int32).reshape(B // G, G * S)
    pooled = _sc_pool(idx, table)
    w2p = jnp.pad(W2, ((0, 0), (0, 128 - W2.shape[1])))
    b2p = jnp.pad(b2, (0, 128 - b2.shape[0])).reshape(1, 128)
    out = _mlp(pooled, W1, b1.reshape(1, HID), w2p, b2p)
    return out[:, :1]


# speedup vs baseline: 1.1827x; 1.1827x over previous
import functools

import jax
import jax.numpy as jnp
from jax import lax
from jax.experimental import pallas as pl
from jax.experimental.pallas import tpu as pltpu
from jax.experimental.pallas import tpu_sc as plsc

B = 16384
S = 50
EMB = 128
HID = 256

NC, NS = 2, 16
NW = NC * NS
EPW = B // NW
G = 1
NGROUPS = EPW // G
GPW = NGROUPS
NVR = EMB // 16
RU = 10
NBUF = 8
DEPTH = 6


def _sc_pool_body(idx_hbm, table_hbm, out_hbm, idx_all, *scratch):
    rows = scratch[0:NBUF]
    outs = scratch[NBUF:2 * NBUF]
    sgs = scratch[2 * NBUF:3 * NBUF]
    sos = scratch[3 * NBUF:4 * NBUF]
    wid = lax.axis_index("s") * NC + lax.axis_index("c")
    ebase = wid * EPW
    pltpu.sync_copy(idx_hbm.at[pl.ds(wid * GPW, GPW)], idx_all)

    def prefetch(g, p):
        pltpu.async_copy(table_hbm.at[idx_all.at[g]], rows[p], sgs[p])

    def consume(i, g, p):
        pltpu.make_async_copy(
            table_hbm.at[idx_all.at[g]], rows[p], sgs[p]).wait()

        @pl.when(i >= 1)
        def _():
            pltpu.make_async_copy(
                outs[p], out_hbm.at[pl.ds(0, G)], sos[p]).wait()

        for e in range(G):
            def row_body(rr, accs, e=e):
                base = e * S + rr * RU
                for k in range(RU):
                    accs = tuple(
                        accs[v] + rows[p][base + k, pl.ds(16 * v, 16)]
                        for v in range(NVR))
                return accs
            accs = lax.fori_loop(
                0, S // RU, row_body,
                tuple(jnp.zeros((16,), jnp.float32) for _ in range(NVR)))
            for v in range(NVR):
                outs[p][e, pl.ds(16 * v, 16)] = accs[v] * (1.0 / S)
        pltpu.async_copy(outs[p], out_hbm.at[pl.ds(ebase + g * G, G)], sos[p])

    for d in range(DEPTH):
        prefetch(d, d)

    def block(i, carry):
        for p in range(NBUF):
            g = NBUF * i + p

            @pl.when(g + DEPTH < NGROUPS)
            def _(p=p, g=g):
                prefetch(g + DEPTH, (p + DEPTH) % NBUF)

            consume(i, g, p)
        return carry

    lax.fori_loop(0, NGROUPS // NBUF, block, 0)
    for p in range(NBUF):
        pltpu.make_async_copy(outs[p], out_hbm.at[pl.ds(0, G)], sos[p]).wait()


_sc_pool = pl.kernel(
    _sc_pool_body,
    out_type=jax.ShapeDtypeStruct((B, EMB), jnp.float32),
    mesh=plsc.VectorSubcoreMesh(core_axis_name="c", subcore_axis_name="s"),
    scratch_types=(
        [pltpu.VMEM((GPW, G * S), jnp.int32)]
        + [pltpu.VMEM((G * S, EMB), jnp.float32) for _ in range(NBUF)]
        + [pltpu.VMEM((G, EMB), jnp.float32) for _ in range(NBUF)]
        + [pltpu.SemaphoreType.DMA for _ in range(2 * NBUF)]
    ),
)


def _mlp_body(x_ref, w1_ref, b1_ref, w2_ref, b2_ref, o_ref):
    x = x_ref[...]
    h = jnp.dot(x, w1_ref[...], preferred_element_type=jnp.float32)
    h = jnp.maximum(h + b1_ref[...], 0.0)
    o = jnp.dot(h, w2_ref[...], preferred_element_type=jnp.float32)
    o_ref[...] = jax.nn.sigmoid(o + b2_ref[...])


def _mlp(x, w1, b1, w2, b2):
    BM = 2048
    grid = (B // BM,)
    return pl.pallas_call(
        _mlp_body,
        out_shape=jax.ShapeDtypeStruct((B, 128), jnp.float32),
        grid=grid,
        in_specs=[
            pl.BlockSpec((BM, EMB), lambda i: (i, 0)),
            pl.BlockSpec((EMB, HID), lambda i: (0, 0)),
            pl.BlockSpec((1, HID), lambda i: (0, 0)),
            pl.BlockSpec((HID, 128), lambda i: (0, 0)),
            pl.BlockSpec((1, 128), lambda i: (0, 0)),
        ],
        out_specs=pl.BlockSpec((BM, 128), lambda i: (i, 0)),
    )(x, w1, b1, w2, b2)


def kernel(inputs, table, W1, b1, W2, b2):
    idx = inputs.astype(jnp.int32).reshape(B // G, G * S)
    pooled = _sc_pool(idx, table)
    w2p = jnp.pad(W2, ((0, 0), (0, 128 - W2.shape[1])))
    b2p = jnp.pad(b2, (0, 128 - b2.shape[0])).reshape(1, 128)
    out = _mlp(pooled, W1, b1.reshape(1, HID), w2p, b2p)
    return out[:, :1]


# speedup vs baseline: 1.1849x; 1.0019x over previous
import functools

import jax
import jax.numpy as jnp
from jax import lax
from jax.experimental import pallas as pl
from jax.experimental.pallas import tpu as pltpu
from jax.experimental.pallas import tpu_sc as plsc

B = 16384
S = 50
EMB = 128
HID = 256

NC, NS = 2, 16
NW = NC * NS
EPW = B // NW
G = 1
NGROUPS = EPW // G
GPW = NGROUPS
NVR = EMB // 16
RU = 10
NBUF = 8
DEPTH = 6


def _sc_pool_body(idx_hbm, table_hbm, out_hbm, idx_all, *scratch):
    rows = scratch[0:NBUF]
    outs = scratch[NBUF:2 * NBUF]
    sgs = scratch[2 * NBUF:3 * NBUF]
    sos = scratch[3 * NBUF:4 * NBUF]
    wid = lax.axis_index("s") * NC + lax.axis_index("c")
    ebase = wid * EPW
    pltpu.sync_copy(idx_hbm.at[pl.ds(wid * GPW, GPW)], idx_all)

    def prefetch(g, p):
        pltpu.async_copy(table_hbm.at[idx_all.at[g]], rows[p], sgs[p])

    def consume(i, g, p):
        pltpu.make_async_copy(
            table_hbm.at[idx_all.at[g]], rows[p], sgs[p]).wait()

        @pl.when(i >= 1)
        def _():
            pltpu.make_async_copy(
                outs[p], out_hbm.at[pl.ds(0, G)], sos[p]).wait()

        for e in range(G):
            def row_body(rr, accs, e=e):
                base = e * S + rr * RU
                for k in range(RU):
                    accs = tuple(
                        accs[v] + rows[p][base + k, pl.ds(16 * v, 16)]
                        for v in range(NVR))
                return accs
            accs = lax.fori_loop(
                0, S // RU, row_body,
                tuple(jnp.zeros((16,), jnp.float32) for _ in range(NVR)))
            for v in range(NVR):
                outs[p][e, pl.ds(16 * v, 16)] = accs[v] * (1.0 / S)
        pltpu.async_copy(outs[p], out_hbm.at[pl.ds(ebase + g * G, G)], sos[p])

    for d in range(DEPTH):
        prefetch(d, d)

    def block(i, carry):
        for p in range(NBUF):
            g = NBUF * i + p

            @pl.when(g + DEPTH < NGROUPS)
            def _(p=p, g=g):
                prefetch(g + DEPTH, (p + DEPTH) % NBUF)

            consume(i, g, p)
        return carry

    lax.fori_loop(0, NGROUPS // NBUF, block, 0)
    for p in range(NBUF):
        pltpu.make_async_copy(outs[p], out_hbm.at[pl.ds(0, G)], sos[p]).wait()


_sc_pool = pl.kernel(
    _sc_pool_body,
    out_type=jax.ShapeDtypeStruct((B, EMB), jnp.float32),
    mesh=plsc.VectorSubcoreMesh(core_axis_name="c", subcore_axis_name="s"),
    scratch_types=(
        [pltpu.VMEM((GPW, G * S), jnp.int32)]
        + [pltpu.VMEM((G * S, EMB), jnp.float32) for _ in range(NBUF)]
        + [pltpu.VMEM((G, EMB), jnp.float32) for _ in range(NBUF)]
        + [pltpu.SemaphoreType.DMA for _ in range(2 * NBUF)]
    ),
)


def _mlp_body(x_ref, w1_ref, b1_ref, w2_ref, b2_ref, o_ref):
    x = x_ref[...]
    h = jnp.dot(x, w1_ref[...], preferred_element_type=jnp.float32)
    h = jnp.maximum(h + b1_ref[...], 0.0)
    o = jnp.dot(h, w2_ref[...], preferred_element_type=jnp.float32)
    o_ref[...] = jax.nn.sigmoid(o + b2_ref[...])[:, :1]


def _mlp(x, w1, b1, w2, b2):
    BM = 2048
    grid = (B // BM,)
    return pl.pallas_call(
        _mlp_body,
        out_shape=jax.ShapeDtypeStruct((B, 1), jnp.float32),
        grid=grid,
        in_specs=[
            pl.BlockSpec((BM, EMB), lambda i: (i, 0)),
            pl.BlockSpec((EMB, HID), lambda i: (0, 0)),
            pl.BlockSpec((1, HID), lambda i: (0, 0)),
            pl.BlockSpec((HID, 128), lambda i: (0, 0)),
            pl.BlockSpec((1, 128), lambda i: (0, 0)),
        ],
        out_specs=pl.BlockSpec((BM, 1), lambda i: (i, 0)),
    )(x, w1, b1, w2, b2)


def kernel(inputs, table, W1, b1, W2, b2):
    idx = inputs.astype(jnp.int32).reshape(B // G, G * S)
    pooled = _sc_pool(idx, table)
    w2p = jnp.pad(W2, ((0, 0), (0, 128 - W2.shape[1])))
    b2p = jnp.pad(b2, (0, 128 - b2.shape[0])).reshape(1, 128)
    return _mlp(pooled, W1, b1.reshape(1, HID), w2p, b2p)
